# baseline (speedup 1.0000x reference)
import jax
import jax.numpy as jnp
import numpy as np
from jax.experimental import pallas as pl
from jax.experimental.pallas import tpu as pltpu

B, C, N, H, W = 4, 4, 8, 512, 512
Hb = 512


def _body(ann_ref, logits_ref, masks_a_ref, masks_b_ref, out_ref, acc_ref):
    i = pl.program_id(0)
    j = pl.program_id(1)
    n_j = pl.num_programs(1)

    @pl.when(jnp.logical_and(i == 0, j == 0))
    def _init_out():
        out_ref[0, 0] = 0.0

    @pl.when(j == 0)
    def _init_acc():
        acc_ref[0, 0] = 0.0

    lg = logits_ref[0, 0]
    ma = masks_a_ref[0, 0]
    mb = masks_b_ref[0, 0]
    covered = (jnp.sum(ma, axis=0) + jnp.sum(mb, axis=0)) > 0.0
    outside = jnp.where(covered, jnp.zeros_like(lg), lg)
    acc_ref[0, 0] += jnp.sum(outside)

    @pl.when(j == n_j - 1)
    def _finish_pair():
        o = acc_ref[0, 0]
        b = i // (C - 1)
        c = i % (C - 1)
        err = jnp.where(o >= 0.0, o * o, 0.0) * ann_ref[b, c + 1]
        out_ref[0, 0] += err


def kernel(logits, box_masks, annotation_mask):
    n_pairs = B * (C - 1)
    grid = (n_pairs, H // Hb)

    out = pl.pallas_call(
        _body,
        grid=grid,
        in_specs=[
            pl.BlockSpec(memory_space=pltpu.SMEM),
            pl.BlockSpec(
                (1, 1, Hb, W),
                lambda i, j: (i // (C - 1), 1 + i % (C - 1), j, 0),
            ),
            pl.BlockSpec(
                (1, 1, N // 2, Hb, W),
                lambda i, j: (i // (C - 1), 1 + i % (C - 1), 0, j, 0),
            ),
            pl.BlockSpec(
                (1, 1, N // 2, Hb, W),
                lambda i, j: (i // (C - 1), 1 + i % (C - 1), 1, j, 0),
            ),
        ],
        out_specs=pl.BlockSpec(memory_space=pltpu.SMEM),
        out_shape=jax.ShapeDtypeStruct((1, 1), jnp.float32),
        scratch_shapes=[pltpu.SMEM((1, 1), jnp.float32)],
    )(annotation_mask, logits, box_masks, box_masks)

    im_size = float(np.prod(logits.shape[2:]))
    return out[0, 0] / im_size
